# baseline (speedup 1.0000x reference)
import functools

import jax
import jax.numpy as jnp
from jax import lax
from jax.experimental import pallas as pl
from jax.experimental.pallas import tpu as pltpu
from jax.experimental.pallas import tpu_sc as plsc

NUM_LAYERS = 12
NUM_TABLES = 3
REPS = NUM_LAYERS // NUM_TABLES


def _sc_gather(idx, w0, w1, w2):
    (n,) = idx.shape
    _, d = w0.shape

    info = plsc.get_sparse_core_info()
    nc, ns = info.num_cores, info.num_subcores
    nw = nc * ns
    tpw = n // nw
    chunk = 32
    nchunk = tpw // chunk
    nsteps = NUM_TABLES * nchunk

    mesh = plsc.VectorSubcoreMesh(core_axis_name="c", subcore_axis_name="s")

    @functools.partial(
        pl.kernel,
        mesh=mesh,
        out_type=jax.ShapeDtypeStruct((REPS, NUM_TABLES, n, d), jnp.float32),
        scratch_types=[
            pltpu.VMEM((tpw,), jnp.int32),
            pltpu.VMEM((chunk, d), jnp.float32),
            pltpu.VMEM((chunk, d), jnp.float32),
            pltpu.SemaphoreType.DMA,
            pltpu.SemaphoreType.DMA,
            pltpu.SemaphoreType.DMA,
            pltpu.SemaphoreType.DMA,
        ],
    )
    def k(idx_hbm, w0_hbm, w1_hbm, w2_hbm, out_hbm, idx_v, buf0, buf1,
          gsem0, gsem1, ssem0, ssem1):
        wid = lax.axis_index("s") * nc + lax.axis_index("c")
        base = wid * tpw
        pltpu.sync_copy(idx_hbm.at[pl.ds(base, tpw)], idx_v)
        tables = (w0_hbm, w1_hbm, w2_hbm)
        bufs = (buf0, buf1)
        gsems = (gsem0, gsem1)
        ssems = (ssem0, ssem1)

        def gather(i):
            t, g = divmod(i, nchunk)
            s = i % 2
            return pltpu.async_copy(
                tables[t].at[idx_v.at[pl.ds(g * chunk, chunk)]], bufs[s], gsems[s]
            )

        def store(i):
            t, g = divmod(i, nchunk)
            s = i % 2
            return pltpu.async_copy(
                bufs[s], out_hbm.at[REPS - 1, t, pl.ds(base + g * chunk, chunk), :],
                ssems[s]
            )

        pending_g = gather(0)
        pending_s = [None, None]
        for i in range(nsteps):
            if i + 1 < nsteps:
                if pending_s[(i + 1) % 2] is not None:
                    pending_s[(i + 1) % 2].wait()
                next_g = gather(i + 1)
            pending_g.wait()
            pending_s[i % 2] = store(i)
            if i + 1 < nsteps:
                pending_g = next_g
        for s in range(2):
            if pending_s[s] is not None:
                pending_s[s].wait()

    return k(idx, w0, w1, w2)


def _tc_broadcast(buf4):
    _, _, n, d = buf4.shape
    ct = 512
    grid = (NUM_TABLES, n // ct)

    def body(in_ref, out_ref):
        x = in_ref[0, 0]
        for r in range(REPS - 1):
            out_ref[r, 0] = x

    return pl.pallas_call(
        body,
        grid=grid,
        in_specs=[
            pl.BlockSpec((1, 1, ct, d), lambda t, c: (REPS - 1, t, c, 0)),
        ],
        out_specs=pl.BlockSpec((REPS - 1, 1, ct, d), lambda t, c: (0, t, c, 0)),
        out_shape=jax.ShapeDtypeStruct(buf4.shape, jnp.float32),
        input_output_aliases={0: 0},
    )(buf4)


def kernel(input_seq, W0, W1, W2):
    b, s = input_seq.shape
    _, d = W0.shape
    idx = input_seq.reshape(b * s)
    buf4 = _sc_gather(idx, W0, W1, W2)
    out = _tc_broadcast(buf4)
    return out.reshape(NUM_LAYERS, b, s, d)


# speedup vs baseline: 1.0016x; 1.0016x over previous
import functools

import jax
import jax.numpy as jnp
from jax import lax
from jax.experimental import pallas as pl
from jax.experimental.pallas import tpu as pltpu
from jax.experimental.pallas import tpu_sc as plsc

NUM_LAYERS = 12
NUM_TABLES = 3
REPS = NUM_LAYERS // NUM_TABLES


def _sc_gather(idx, w0, w1, w2):
    (n,) = idx.shape
    _, d = w0.shape

    info = plsc.get_sparse_core_info()
    nc, ns = info.num_cores, info.num_subcores
    nw = nc * ns
    tpw = n // nw
    chunk = 32
    nchunk = tpw // chunk
    nsteps = NUM_TABLES * nchunk

    mesh = plsc.VectorSubcoreMesh(core_axis_name="c", subcore_axis_name="s")

    @functools.partial(
        pl.kernel,
        mesh=mesh,
        out_type=jax.ShapeDtypeStruct((REPS, NUM_TABLES, n, d), jnp.float32),
        scratch_types=[
            pltpu.VMEM((tpw,), jnp.int32),
            pltpu.VMEM((chunk, d), jnp.float32),
            pltpu.VMEM((chunk, d), jnp.float32),
            pltpu.SemaphoreType.DMA,
            pltpu.SemaphoreType.DMA,
            pltpu.SemaphoreType.DMA,
            pltpu.SemaphoreType.DMA,
        ],
    )
    def k(idx_hbm, w0_hbm, w1_hbm, w2_hbm, out_hbm, idx_v, buf0, buf1,
          gsem0, gsem1, ssem0, ssem1):
        wid = lax.axis_index("s") * nc + lax.axis_index("c")
        base = wid * tpw
        pltpu.sync_copy(idx_hbm.at[pl.ds(base, tpw)], idx_v)
        tables = (w0_hbm, w1_hbm, w2_hbm)
        bufs = (buf0, buf1)
        gsems = (gsem0, gsem1)
        ssems = (ssem0, ssem1)

        def gather(i):
            t, g = divmod(i, nchunk)
            s = i % 2
            return pltpu.async_copy(
                tables[t].at[idx_v.at[pl.ds(g * chunk, chunk)]], bufs[s], gsems[s]
            )

        def store(i):
            t, g = divmod(i, nchunk)
            s = i % 2
            return pltpu.async_copy(
                bufs[s], out_hbm.at[REPS - 1, t, pl.ds(base + g * chunk, chunk), :],
                ssems[s]
            )

        pending_g = gather(0)
        pending_s = [None, None]
        for i in range(nsteps):
            if i + 1 < nsteps:
                if pending_s[(i + 1) % 2] is not None:
                    pending_s[(i + 1) % 2].wait()
                next_g = gather(i + 1)
            pending_g.wait()
            pending_s[i % 2] = store(i)
            if i + 1 < nsteps:
                pending_g = next_g
        for s in range(2):
            if pending_s[s] is not None:
                pending_s[s].wait()

    return k(idx, w0, w1, w2)


def _tc_broadcast(buf4):
    _, _, n, d = buf4.shape
    ct = 1024
    grid = (NUM_TABLES, n // ct)

    def body(in_ref, out_ref, sem):
        t = pl.program_id(0)
        c = pl.program_id(1)
        copies = [
            pltpu.make_async_copy(
                in_ref,
                out_ref.at[pl.ds(r, 1), pl.ds(t, 1), pl.ds(c * ct, ct), :],
                sem,
            )
            for r in range(REPS - 1)
        ]
        for cp in copies:
            cp.start()
        for cp in copies:
            cp.wait()

    return pl.pallas_call(
        body,
        grid=grid,
        in_specs=[
            pl.BlockSpec((1, 1, ct, d), lambda t, c: (REPS - 1, t, c, 0)),
        ],
        out_specs=pl.BlockSpec(memory_space=pl.ANY),
        out_shape=jax.ShapeDtypeStruct(buf4.shape, jnp.float32),
        scratch_shapes=[pltpu.SemaphoreType.DMA],
        input_output_aliases={0: 0},
    )(buf4)


def kernel(input_seq, W0, W1, W2):
    b, s = input_seq.shape
    _, d = W0.shape
    idx = input_seq.reshape(b * s)
    buf4 = _sc_gather(idx, W0, W1, W2)
    out = _tc_broadcast(buf4)
    return out.reshape(NUM_LAYERS, b, s, d)


# speedup vs baseline: 1.1193x; 1.1175x over previous
import functools

import jax
import jax.numpy as jnp
from jax import lax
from jax.experimental import pallas as pl
from jax.experimental.pallas import tpu as pltpu
from jax.experimental.pallas import tpu_sc as plsc

NUM_LAYERS = 12
NUM_TABLES = 3
REPS = NUM_LAYERS // NUM_TABLES


def _sc_lookup(idx, w0, w1, w2):
    (n,) = idx.shape
    _, d = w0.shape

    info = plsc.get_sparse_core_info()
    nc, ns = info.num_cores, info.num_subcores
    nw = nc * ns
    tpw = n // nw
    chunk = 32
    nchunk = tpw // chunk
    nsteps = NUM_TABLES * nchunk

    mesh = plsc.VectorSubcoreMesh(core_axis_name="c", subcore_axis_name="s")

    @functools.partial(
        pl.kernel,
        mesh=mesh,
        out_type=jax.ShapeDtypeStruct((NUM_LAYERS, n, d), jnp.float32),
        scratch_types=[
            pltpu.VMEM((tpw,), jnp.int32),
            pltpu.VMEM((chunk, d), jnp.float32),
            pltpu.VMEM((chunk, d), jnp.float32),
            pltpu.SemaphoreType.DMA,
            pltpu.SemaphoreType.DMA,
            pltpu.SemaphoreType.DMA,
            pltpu.SemaphoreType.DMA,
        ],
    )
    def k(idx_hbm, w0_hbm, w1_hbm, w2_hbm, out_hbm, idx_v, buf0, buf1,
          gsem0, gsem1, ssem0, ssem1):
        wid = lax.axis_index("s") * nc + lax.axis_index("c")
        base = wid * tpw
        pltpu.sync_copy(idx_hbm.at[pl.ds(base, tpw)], idx_v)
        tables = (w0_hbm, w1_hbm, w2_hbm)
        bufs = (buf0, buf1)
        gsems = (gsem0, gsem1)
        ssems = (ssem0, ssem1)

        def gather(i):
            t, g = divmod(i, nchunk)
            s = i % 2
            return pltpu.async_copy(
                tables[t].at[idx_v.at[pl.ds(g * chunk, chunk)]], bufs[s], gsems[s]
            )

        def stores(i):
            t, g = divmod(i, nchunk)
            s = i % 2
            return [
                pltpu.async_copy(
                    bufs[s],
                    out_hbm.at[t + NUM_TABLES * r, pl.ds(base + g * chunk, chunk), :],
                    ssems[s],
                )
                for r in range(REPS)
            ]

        pending_g = gather(0)
        pending_s = [None, None]
        for i in range(nsteps):
            if i + 1 < nsteps:
                if pending_s[(i + 1) % 2] is not None:
                    for c in pending_s[(i + 1) % 2]:
                        c.wait()
                next_g = gather(i + 1)
            pending_g.wait()
            pending_s[i % 2] = stores(i)
            if i + 1 < nsteps:
                pending_g = next_g
        for s in range(2):
            if pending_s[s] is not None:
                for c in pending_s[s]:
                    c.wait()

    return k(idx, w0, w1, w2)


def kernel(input_seq, W0, W1, W2):
    b, s = input_seq.shape
    _, d = W0.shape
    idx = input_seq.reshape(b * s)
    out = _sc_lookup(idx, W0, W1, W2)
    return out.reshape(NUM_LAYERS, b, s, d)
